# baseline (speedup 1.0000x reference)
import functools

import jax
import jax.numpy as jnp
from jax import lax
from jax.experimental import pallas as pl
from jax.experimental.pallas import tpu as pltpu
from jax.experimental.pallas import tpu_sc as plsc

B = 32768
D = 4096
E = 64
K = 8
TB = 512

NC, NS, L = 2, 16, 16
NW = NC * NS
TOKW = B // NW
CH = 2


def _router_body(f_ref, w_ref, b_ref, idx_ref, wt_ref):
    f = f_ref[...]
    st = jax.lax.dot_general(
        w_ref[...], f, (((1,), (1,)), ((), ())), preferred_element_type=jnp.float32
    ) + b_ref[...]

    iota_e = lax.broadcasted_iota(jnp.int32, (E, TB), 0)
    work = st
    vals, idxs = [], []
    for _ in range(K):
        m = jnp.max(work, axis=0, keepdims=True)
        sel = work >= m
        am = jnp.min(jnp.where(sel, iota_e, E), axis=0, keepdims=True)
        vals.append(m)
        idxs.append(am)
        work = jnp.where(sel, -jnp.inf, work)
    v = jnp.concatenate(vals, axis=0)
    ix = jnp.concatenate(idxs, axis=0)
    ex = jnp.exp(v - v[0:1])
    wt_ref[...] = ex / jnp.sum(ex, axis=0, keepdims=True)
    idx_ref[...] = ix


def _tc_router(features, W_attn, b_attn):
    grid = (B // TB,)
    return pl.pallas_call(
        _router_body,
        grid=grid,
        in_specs=[
            pl.BlockSpec((TB, D), lambda i: (i, 0)),
            pl.BlockSpec((E, D), lambda i: (0, 0)),
            pl.BlockSpec((E, 1), lambda i: (0, 0)),
        ],
        out_specs=[
            pl.BlockSpec((K, TB), lambda i: (0, i)),
            pl.BlockSpec((K, TB), lambda i: (0, i)),
        ],
        out_shape=[
            jax.ShapeDtypeStruct((K, B), jnp.int32),
            jax.ShapeDtypeStruct((K, B), jnp.float32),
        ],
        compiler_params=pltpu.CompilerParams(
            dimension_semantics=("arbitrary",),
        ),
    )(features, W_attn, b_attn.reshape(E, 1))


CW = 512


def _sc_combine_body(feat_hbm, idx_hbm, w_hbm, lora_hbm, out_hbm,
                     idx_v, w_v, rows_v, f_v, o_v, sem):
    wid = lax.axis_index("s") * NC + lax.axis_index("c")
    base_tok = wid * TOKW

    def chunk(c, carry):
        tok = base_tok + c * CH
        pltpu.sync_copy(idx_hbm.at[pl.ds(tok * K, CH * K)], idx_v)
        pltpu.sync_copy(w_hbm.at[pl.ds(tok * K, CH * K)], w_v)
        pltpu.async_copy(lora_hbm.at[idx_v], rows_v, sem).wait()
        pltpu.sync_copy(feat_hbm.at[pl.ds(tok, CH)], f_v)
        w_all = w_v[...]

        def col(j, carry2):
            sl = pl.ds(j * CW, CW)
            for t in range(CH):
                acc = f_v[t, sl]
                for k in range(K):
                    acc = acc + w_all[t * K + k] * rows_v[t * K + k, sl]
                o_v[t, sl] = acc
            return carry2

        lax.fori_loop(0, D // CW, col, 0)
        pltpu.sync_copy(o_v, out_hbm.at[pl.ds(tok, CH)])
        return carry

    lax.fori_loop(0, TOKW // CH, chunk, 0)


def _sc_combine(features, idx_flat, w_flat, lora_ranks):
    mesh = plsc.VectorSubcoreMesh(core_axis_name="c", subcore_axis_name="s")
    return pl.kernel(
        _sc_combine_body,
        mesh=mesh,
        out_type=jax.ShapeDtypeStruct((B, D), jnp.float32),
        scratch_types=[
            pltpu.VMEM((CH * K,), jnp.int32),
            pltpu.VMEM((CH * K,), jnp.float32),
            pltpu.VMEM((CH * K, D), jnp.float32),
            pltpu.VMEM((CH, D), jnp.float32),
            pltpu.VMEM((CH, D), jnp.float32),
            pltpu.SemaphoreType.DMA,
        ],
    )(features, idx_flat, w_flat, lora_ranks)


@jax.jit
def kernel(features, W_attn, b_attn, lora_ranks):
    ix, wt = _tc_router(features, W_attn, b_attn)
    idx_flat = ix.T.reshape(B * K)
    w_flat = wt.T.reshape(B * K)
    return _sc_combine(features, idx_flat, w_flat, lora_ranks)


# speedup vs baseline: 17.2625x; 17.2625x over previous
import jax
import jax.numpy as jnp
from jax.experimental import pallas as pl
from jax.experimental.pallas import tpu as pltpu

B = 32768
D = 4096
E = 64
K = 8
TB = 512


def _router_body(f_ref, w_ref, b_ref, lora_ref, o_ref):
    f = f_ref[...]
    st = jax.lax.dot_general(
        w_ref[...], f, (((1,), (1,)), ((), ())), preferred_element_type=jnp.float32
    ) + b_ref[...]

    work = st
    m = None
    for _ in range(K):
        m = jnp.max(work, axis=0, keepdims=True)
        work = jnp.where(work >= m, -jnp.inf, work)
    mask = st >= m

    mx = jnp.max(jnp.where(mask, st, -jnp.inf), axis=0, keepdims=True)
    ex = jnp.where(mask, jnp.exp(st - mx), 0.0)
    wts = ex / jnp.sum(ex, axis=0, keepdims=True)

    combined = jax.lax.dot_general(
        wts, lora_ref[...], (((0,), (0,)), ((), ())),
        preferred_element_type=jnp.float32,
    )
    o_ref[...] = f + combined


@jax.jit
def kernel(features, W_attn, b_attn, lora_ranks):
    b2 = b_attn.reshape(E, 1)
    grid = (B // TB,)
    return pl.pallas_call(
        _router_body,
        grid=grid,
        in_specs=[
            pl.BlockSpec((TB, D), lambda i: (i, 0)),
            pl.BlockSpec((E, D), lambda i: (0, 0)),
            pl.BlockSpec((E, 1), lambda i: (0, 0)),
            pl.BlockSpec((E, D), lambda i: (0, 0)),
        ],
        out_specs=pl.BlockSpec((TB, D), lambda i: (i, 0)),
        out_shape=jax.ShapeDtypeStruct((B, D), jnp.float32),
        compiler_params=pltpu.CompilerParams(
            dimension_semantics=("parallel",),
        ),
    )(features, W_attn, b2, lora_ranks)
